# baseline (device time: 106226 ns/iter reference)
import jax
import jax.numpy as jnp
from jax import lax
from jax.experimental import pallas as pl
from jax.experimental.pallas import tpu as pltpu

N_DEV = 16
H = N_DEV // 2
NB = 2

RING = [0, 4, 8, 12, 13, 9, 5, 1, 2, 6, 10, 14, 15, 11, 7, 3]
POS = [RING.index(i) for i in range(N_DEV)]


def kernel(x, w_mat):
    m, k_per = x.shape
    n = w_mat.shape[1]
    chunk = m // N_DEV
    n2 = n // NB

    def _mod(a):
        return lax.rem(a + 2 * N_DEV, N_DEV)

    def body(ring_ref, x_ref, w_ref, out_ref, part_bf, cw_send, ccw_send,
             rs_cw, rs_ccw, ag_cw, ag_ccw,
             cw_send_sems, ccw_send_sems,
             rs_cw_sems, rs_ccw_sems, ag_cw_sems, ag_ccw_sems):
        my = ring_ref[0]
        left = ring_ref[1]
        right = ring_ref[2]

        def rows_of(c):
            return pl.ds(c * chunk, chunk)

        def cols_of(b):
            return pl.ds(b * n2, n2)

        def rs_cw_rdma(s, b):
            return pltpu.make_async_remote_copy(
                src_ref=cw_send.at[:, cols_of(b)],
                dst_ref=rs_cw.at[s, :, cols_of(b)],
                send_sem=cw_send_sems.at[b],
                recv_sem=rs_cw_sems.at[s, b],
                device_id=(right,),
                device_id_type=pl.DeviceIdType.MESH,
            )

        def rs_ccw_rdma(s, b):
            return pltpu.make_async_remote_copy(
                src_ref=ccw_send.at[:, cols_of(b)],
                dst_ref=rs_ccw.at[s, :, cols_of(b)],
                send_sem=ccw_send_sems.at[b],
                recv_sem=rs_ccw_sems.at[s, b],
                device_id=(left,),
                device_id_type=pl.DeviceIdType.MESH,
            )

        def ag_cw_rdma(t, b):
            return pltpu.make_async_remote_copy(
                src_ref=(cw_send.at[:, cols_of(b)] if t == 0
                         else ag_cw.at[t - 1, :, cols_of(b)]),
                dst_ref=ag_cw.at[t, :, cols_of(b)],
                send_sem=cw_send_sems.at[b],
                recv_sem=ag_cw_sems.at[t, b],
                device_id=(right,),
                device_id_type=pl.DeviceIdType.MESH,
            )

        def ag_ccw_rdma(t, b):
            return pltpu.make_async_remote_copy(
                src_ref=(ccw_send.at[:, cols_of(b)] if t == 0
                         else ag_ccw.at[t - 1, :, cols_of(b)]),
                dst_ref=ag_ccw.at[t, :, cols_of(b)],
                send_sem=ccw_send_sems.at[b],
                recv_sem=ag_ccw_sems.at[t, b],
                device_id=(left,),
                device_id_type=pl.DeviceIdType.MESH,
            )

        def rs_cw_fwd(s, b):
            return pltpu.make_async_remote_copy(
                src_ref=(cw_send.at[:, cols_of(b)] if (s == 1 and b == 1)
                         else rs_cw.at[s - 1, :, cols_of(b)]),
                dst_ref=rs_cw.at[s, :, cols_of(b)],
                send_sem=cw_send_sems.at[b],
                recv_sem=rs_cw_sems.at[s, b],
                device_id=(right,),
                device_id_type=pl.DeviceIdType.MESH,
            )

        def rs_ccw_fwd(s, b):
            return pltpu.make_async_remote_copy(
                src_ref=(ccw_send.at[:, cols_of(b)] if (s == 1 and b == 0)
                         else rs_ccw.at[s - 1, :, cols_of(b)]),
                dst_ref=rs_ccw.at[s, :, cols_of(b)],
                send_sem=ccw_send_sems.at[b],
                recv_sem=rs_ccw_sems.at[s, b],
                device_id=(left,),
                device_id_type=pl.DeviceIdType.MESH,
            )

        barrier_sem = pltpu.get_barrier_semaphore()
        for nbr in (left, right):
            pl.semaphore_signal(
                barrier_sem, inc=1,
                device_id=(nbr,), device_id_type=pl.DeviceIdType.MESH,
            )
        pl.semaphore_wait(barrier_sem, 2)

        a = _mod(my + H)
        part = jnp.dot(
            x_ref[rows_of(a), :], w_ref[:, :],
            preferred_element_type=jnp.float32,
        )
        cw_send[:, cols_of(0)] = part[:, :n2].astype(jnp.bfloat16)
        rs_cw_rdma(0, 0).start()
        ccw_send[:, cols_of(1)] = part[:, n2:].astype(jnp.bfloat16)
        rs_ccw_rdma(0, 1).start()

        out_ref[:, :] = jnp.dot(
            x_ref[:, :], w_ref[:, :], preferred_element_type=jnp.float32
        )
        part_bf[:, :] = out_ref[:, :].astype(jnp.bfloat16)

        for s in range(1, H):
            c_cw = _mod(my + H - s)
            c_ccw = _mod(my - H + s)
            for b in range(NB):
                if not (s == 1 and b == 1):
                    d = rs_cw_rdma(s - 1, b)
                    d.wait_recv()
                    d.wait_send()
                rs_cw_fwd(s, b).start()
                if not (s == 1 and b == 0):
                    d = rs_ccw_rdma(s - 1, b)
                    d.wait_recv()
                    d.wait_send()
                rs_ccw_fwd(s, b).start()

        rows = rows_of(my)
        for b in range(NB):
            dcw = rs_cw_rdma(H - 1, b)
            dcw.wait_recv()
            dccw = rs_ccw_rdma(H - 1, b)
            dccw.wait_recv()
            dcw.wait_send()
            dccw.wait_send()
            ag_cw_rdma(0, b).start()
            ag_ccw_rdma(0, b).start()

        for t in range(1, H):
            for b in range(NB):
                d = ag_cw_rdma(t - 1, b)
                d.wait_recv()
                d.wait_send()
                if not (t == H - 1 and b == 1):
                    ag_cw_rdma(t, b).start()
                d = ag_ccw_rdma(t - 1, b)
                d.wait_recv()
                d.wait_send()
                if not (t == H - 1 and b == 0):
                    ag_ccw_rdma(t, b).start()

        d = ag_cw_rdma(H - 1, 0)
        d.wait_recv()
        out_ref[rows_of(_mod(my - H)), cols_of(0)] = (
            ag_cw[H - 1, :, cols_of(0)].astype(jnp.float32)
        )
        d.wait_send()
        d = ag_ccw_rdma(H - 1, 1)
        d.wait_recv()
        out_ref[rows_of(_mod(my - H)), cols_of(1)] = (
            ag_ccw[H - 1, :, cols_of(1)].astype(jnp.float32)
        )
        d.wait_send()

    my_l = lax.axis_index("i")
    ring = jnp.array(RING, dtype=jnp.int32)
    pos = jnp.array(POS, dtype=jnp.int32)
    k = pos[my_l]
    right_l = ring[jnp.remainder(k + 1, N_DEV)]
    left_l = ring[jnp.remainder(k - 1 + N_DEV, N_DEV)]
    ring_info = jnp.stack([k, left_l, right_l]).astype(jnp.int32)

    return pl.pallas_call(
        body,
        out_shape=jax.ShapeDtypeStruct((m, n), jnp.float32),
        in_specs=[
            pl.BlockSpec(memory_space=pltpu.SMEM),
            pl.BlockSpec(memory_space=pltpu.VMEM),
            pl.BlockSpec(memory_space=pltpu.VMEM),
        ],
        out_specs=pl.BlockSpec(memory_space=pltpu.VMEM),
        scratch_shapes=[
            pltpu.VMEM((m, n), jnp.bfloat16),
            pltpu.VMEM((chunk, n), jnp.bfloat16),
            pltpu.VMEM((chunk, n), jnp.bfloat16),
            pltpu.VMEM((H, chunk, n), jnp.bfloat16),
            pltpu.VMEM((H, chunk, n), jnp.bfloat16),
            pltpu.VMEM((H, chunk, n), jnp.bfloat16),
            pltpu.VMEM((H, chunk, n), jnp.bfloat16),
            pltpu.SemaphoreType.DMA((NB,)),
            pltpu.SemaphoreType.DMA((NB,)),
            pltpu.SemaphoreType.DMA((H, NB)),
            pltpu.SemaphoreType.DMA((H, NB)),
            pltpu.SemaphoreType.DMA((H, NB)),
            pltpu.SemaphoreType.DMA((H, NB)),
        ],
        compiler_params=pltpu.CompilerParams(collective_id=0),
    )(ring_info, x, w_mat)


# device time: 101413 ns/iter; 1.0475x vs baseline; 1.0475x over previous
import jax
import jax.numpy as jnp
from jax import lax
from jax.experimental import pallas as pl
from jax.experimental.pallas import tpu as pltpu

N_DEV = 16
H = N_DEV // 2
NB = 2

RING = [0, 4, 8, 12, 13, 9, 5, 1, 2, 6, 10, 14, 15, 11, 7, 3]
POS = [RING.index(i) for i in range(N_DEV)]


def kernel(x, w_mat):
    m, k_per = x.shape
    n = w_mat.shape[1]
    chunk = m // N_DEV
    n2 = n // NB

    def _mod(a):
        return lax.rem(a + 2 * N_DEV, N_DEV)

    def _pos_of(l):
        z = l // 4
        p = l % 4
        return jnp.where(
            p == 0, z,
            jnp.where(p == 1, 7 - z, jnp.where(p == 2, 8 + z, 15 - z)),
        )

    def _ring_at(k):
        q = k // 4
        r = k % 4
        return jnp.where(
            q == 0, 4 * r,
            jnp.where(
                q == 1, 4 * (3 - r) + 1,
                jnp.where(q == 2, 4 * r + 2, 4 * (3 - r) + 3),
            ),
        )

    def body(x_ref, w_ref, out_ref, part_bf, cw_send, ccw_send,
             rs_cw, rs_ccw, ag_cw, ag_ccw,
             cw_send_sems, ccw_send_sems,
             rs_cw_sems, rs_ccw_sems, ag_cw_sems, ag_ccw_sems):
        my_l = lax.axis_index("i")
        my = _pos_of(my_l)
        left = _ring_at(_mod(my - 1))
        right = _ring_at(_mod(my + 1))

        def rows_of(c):
            return pl.ds(c * chunk, chunk)

        def cols_of(b):
            return pl.ds(b * n2, n2)

        def rs_cw_rdma(s, b):
            return pltpu.make_async_remote_copy(
                src_ref=cw_send.at[:, cols_of(b)],
                dst_ref=rs_cw.at[s, :, cols_of(b)],
                send_sem=cw_send_sems.at[b],
                recv_sem=rs_cw_sems.at[s, b],
                device_id=(right,),
                device_id_type=pl.DeviceIdType.MESH,
            )

        def rs_ccw_rdma(s, b):
            return pltpu.make_async_remote_copy(
                src_ref=ccw_send.at[:, cols_of(b)],
                dst_ref=rs_ccw.at[s, :, cols_of(b)],
                send_sem=ccw_send_sems.at[b],
                recv_sem=rs_ccw_sems.at[s, b],
                device_id=(left,),
                device_id_type=pl.DeviceIdType.MESH,
            )

        def ag_cw_rdma(t, b):
            return pltpu.make_async_remote_copy(
                src_ref=(cw_send.at[:, cols_of(b)] if t == 0
                         else ag_cw.at[t - 1, :, cols_of(b)]),
                dst_ref=ag_cw.at[t, :, cols_of(b)],
                send_sem=cw_send_sems.at[b],
                recv_sem=ag_cw_sems.at[t, b],
                device_id=(right,),
                device_id_type=pl.DeviceIdType.MESH,
            )

        def ag_ccw_rdma(t, b):
            return pltpu.make_async_remote_copy(
                src_ref=(ccw_send.at[:, cols_of(b)] if t == 0
                         else ag_ccw.at[t - 1, :, cols_of(b)]),
                dst_ref=ag_ccw.at[t, :, cols_of(b)],
                send_sem=ccw_send_sems.at[b],
                recv_sem=ag_ccw_sems.at[t, b],
                device_id=(left,),
                device_id_type=pl.DeviceIdType.MESH,
            )

        barrier_sem = pltpu.get_barrier_semaphore()
        for nbr in (left, right):
            pl.semaphore_signal(
                barrier_sem, inc=1,
                device_id=(nbr,), device_id_type=pl.DeviceIdType.MESH,
            )
        pl.semaphore_wait(barrier_sem, 2)

        a = _mod(my + H)
        part = jnp.dot(
            x_ref[rows_of(a), :], w_ref[:, :],
            preferred_element_type=jnp.float32,
        )
        cw_send[:, cols_of(0)] = part[:, :n2].astype(jnp.bfloat16)
        rs_cw_rdma(0, 0).start()
        ccw_send[:, cols_of(1)] = part[:, n2:].astype(jnp.bfloat16)
        rs_ccw_rdma(0, 1).start()

        out_ref[:, :] = jnp.dot(
            x_ref[:, :], w_ref[:, :], preferred_element_type=jnp.float32
        )
        part_bf[:, :] = out_ref[:, :].astype(jnp.bfloat16)

        for s in range(1, H):
            c_cw = _mod(my + H - s)
            c_ccw = _mod(my - H + s)
            for b in range(NB):
                if s == 1 and b == 1:
                    cw_send[:, cols_of(1)] = part_bf[rows_of(c_cw), cols_of(1)]
                else:
                    d = rs_cw_rdma(s - 1, b)
                    d.wait_recv()
                    d.wait_send()
                    cw_send[:, cols_of(b)] = (
                        rs_cw[s - 1, :, cols_of(b)]
                        + part_bf[rows_of(c_cw), cols_of(b)]
                    )
                rs_cw_rdma(s, b).start()

                if s == 1 and b == 0:
                    ccw_send[:, cols_of(0)] = part_bf[rows_of(c_ccw), cols_of(0)]
                else:
                    d = rs_ccw_rdma(s - 1, b)
                    d.wait_recv()
                    d.wait_send()
                    ccw_send[:, cols_of(b)] = (
                        rs_ccw[s - 1, :, cols_of(b)]
                        + part_bf[rows_of(c_ccw), cols_of(b)]
                    )
                rs_ccw_rdma(s, b).start()

        rows = rows_of(my)
        for b in range(NB):
            dcw = rs_cw_rdma(H - 1, b)
            dcw.wait_recv()
            dccw = rs_ccw_rdma(H - 1, b)
            dccw.wait_recv()
            z = (
                out_ref[rows, cols_of(b)]
                + rs_cw[H - 1, :, cols_of(b)].astype(jnp.float32)
                + rs_ccw[H - 1, :, cols_of(b)].astype(jnp.float32)
            )
            z = z * (1.0 / (1.0 + jnp.exp(-z)))
            out_ref[rows, cols_of(b)] = z
            dcw.wait_send()
            dccw.wait_send()
            zb = z.astype(jnp.bfloat16)
            cw_send[:, cols_of(b)] = zb
            ccw_send[:, cols_of(b)] = zb
            ag_cw_rdma(0, b).start()
            ag_ccw_rdma(0, b).start()

        for t in range(1, H):
            for b in range(NB):
                d = ag_cw_rdma(t - 1, b)
                d.wait_recv()
                d.wait_send()
                if not (t == H - 1 and b == 1):
                    ag_cw_rdma(t, b).start()
                d = ag_ccw_rdma(t - 1, b)
                d.wait_recv()
                d.wait_send()
                if not (t == H - 1 and b == 0):
                    ag_ccw_rdma(t, b).start()
            for b in range(NB):
                out_ref[rows_of(_mod(my - t)), cols_of(b)] = (
                    ag_cw[t - 1, :, cols_of(b)].astype(jnp.float32)
                )
                out_ref[rows_of(_mod(my + t)), cols_of(b)] = (
                    ag_ccw[t - 1, :, cols_of(b)].astype(jnp.float32)
                )

        d = ag_cw_rdma(H - 1, 0)
        d.wait_recv()
        out_ref[rows_of(_mod(my - H)), cols_of(0)] = (
            ag_cw[H - 1, :, cols_of(0)].astype(jnp.float32)
        )
        d.wait_send()
        d = ag_ccw_rdma(H - 1, 1)
        d.wait_recv()
        out_ref[rows_of(_mod(my - H)), cols_of(1)] = (
            ag_ccw[H - 1, :, cols_of(1)].astype(jnp.float32)
        )
        d.wait_send()

    return pl.pallas_call(
        body,
        out_shape=jax.ShapeDtypeStruct((m, n), jnp.float32),
        in_specs=[
            pl.BlockSpec(memory_space=pltpu.VMEM),
            pl.BlockSpec(memory_space=pltpu.VMEM),
        ],
        out_specs=pl.BlockSpec(memory_space=pltpu.VMEM),
        scratch_shapes=[
            pltpu.VMEM((m, n), jnp.bfloat16),
            pltpu.VMEM((chunk, n), jnp.bfloat16),
            pltpu.VMEM((chunk, n), jnp.bfloat16),
            pltpu.VMEM((H, chunk, n), jnp.bfloat16),
            pltpu.VMEM((H, chunk, n), jnp.bfloat16),
            pltpu.VMEM((H, chunk, n), jnp.bfloat16),
            pltpu.VMEM((H, chunk, n), jnp.bfloat16),
            pltpu.SemaphoreType.DMA((NB,)),
            pltpu.SemaphoreType.DMA((NB,)),
            pltpu.SemaphoreType.DMA((H, NB)),
            pltpu.SemaphoreType.DMA((H, NB)),
            pltpu.SemaphoreType.DMA((H, NB)),
            pltpu.SemaphoreType.DMA((H, NB)),
        ],
        compiler_params=pltpu.CompilerParams(collective_id=0),
    )(x, w_mat)
